# baseline (speedup 1.0000x reference)
import functools

import jax
import jax.numpy as jnp
from jax import lax
from jax.experimental import pallas as pl
from jax.experimental.pallas import tpu as pltpu
from jax.experimental.pallas import tpu_sc as plsc

GRID = 32
NUM_VOXELS = GRID ** 3

_B, _D, _N = 4, 128, 100000
_NW = 32
_CH = 3136
_CH_LAST = _N - (_NW - 1) * _CH

_NR = 64
_RV = NUM_VOXELS // _NR
_SCH = 4096
_NCH = 25
_SCH_LAST = _N - (_NCH - 1) * _SCH
_RB = 256
_PACK = 1 << 19

_SC_PARAMS = pltpu.CompilerParams(needs_layout_passes=False)


def _make_idx_kernel():
    mesh = plsc.VectorSubcoreMesh(core_axis_name="c", subcore_axis_name="s")

    @functools.partial(
        pl.kernel,
        mesh=mesh,
        compiler_params=_SC_PARAMS,
        out_type=jax.ShapeDtypeStruct((_B * _N,), jnp.int32),
        scratch_types=[
            pltpu.VMEM((_CH * 3,), jnp.float32),
            pltpu.VMEM((_CH,), jnp.int32),
        ],
    )
    def idx_kernel(xyz_hbm, idx_hbm, xyz_v, out_v):
        w = lax.axis_index("s") * 2 + lax.axis_index("c")
        start = w * _CH
        lanes = lax.iota(jnp.int32, 16)
        nsteps = jnp.where(w == _NW - 1, _CH_LAST // 16, _CH // 16)

        for b in range(_B):
            @pl.when(w == _NW - 1)
            def _():
                pltpu.sync_copy(
                    xyz_hbm.at[pl.ds(b * _N * 3 + start * 3, _CH_LAST * 3)],
                    xyz_v.at[pl.ds(0, _CH_LAST * 3)])

            @pl.when(w < _NW - 1)
            def _():
                pltpu.sync_copy(
                    xyz_hbm.at[pl.ds(b * _N * 3 + start * 3, _CH * 3)], xyz_v)

            def step(g, carry):
                pos = (g * 16 + lanes) * 3
                x = plsc.load_gather(xyz_v, [pos])
                y = plsc.load_gather(xyz_v, [pos + 1])
                z = plsc.load_gather(xyz_v, [pos + 2])

                def vox(v):
                    vi = (v * float(GRID)).astype(jnp.int32)
                    return jnp.clip(vi, 0, GRID - 1)

                flat = vox(x) * (GRID * GRID) + vox(y) * GRID + vox(z)
                plsc.store_scatter(out_v, [g * 16 + lanes], flat)
                return carry

            lax.fori_loop(0, nsteps, step, 0)

            @pl.when(w == _NW - 1)
            def _():
                pltpu.sync_copy(out_v.at[pl.ds(0, _CH_LAST)],
                                idx_hbm.at[pl.ds(b * _N + start, _CH_LAST)])

            @pl.when(w < _NW - 1)
            def _():
                pltpu.sync_copy(out_v, idx_hbm.at[pl.ds(b * _N + start, _CH)])

    return idx_kernel


_idx_kernel = _make_idx_kernel()


_TRN = 2048


def _tr_body(f_ref, o_ref):
    o_ref[0] = jnp.swapaxes(f_ref[0], 0, 1)


def _transpose_features(features):
    nblk = (_N + _TRN - 1) // _TRN
    return pl.pallas_call(
        _tr_body,
        out_shape=jax.ShapeDtypeStruct((_B, _N, _D), jnp.float32),
        grid=(_B, nblk),
        in_specs=[pl.BlockSpec((1, _D, _TRN), lambda b, i: (b, 0, i))],
        out_specs=pl.BlockSpec((1, _TRN, _D), lambda b, i: (b, i, 0)),
    )(features)


def _make_scatter_kernel():
    mesh = plsc.VectorSubcoreMesh(core_axis_name="c", subcore_axis_name="s")

    @functools.partial(
        pl.kernel,
        mesh=mesh,
        compiler_params=_SC_PARAMS,
        out_type=(
            jax.ShapeDtypeStruct((_B * NUM_VOXELS * _D,), jnp.float32),
            jax.ShapeDtypeStruct((_B * NUM_VOXELS,), jnp.int32),
        ),
        scratch_types=[
            pltpu.VMEM((_RV * _D,), jnp.float32),
            pltpu.VMEM((_SCH,), jnp.int32),
            pltpu.VMEM((_SCH,), jnp.int32),
            pltpu.VMEM((_RB,), jnp.int32),
            pltpu.VMEM((_RB,), jnp.int32),
            pltpu.VMEM((_RB, _D), jnp.float32),
            pltpu.VMEM((_RV,), jnp.int32),
            pltpu.SemaphoreType.DMA,
        ],
    )
    def scat(ft_hbm, idx_hbm, vf_hbm, cnt_hbm,
             tbl, idxb, selp, gid, goff, rows, cnts, gsem):
        w = lax.axis_index("s") * 2 + lax.axis_index("c")
        lanes = lax.iota(jnp.int32, 16)
        minf = jnp.full((16,), -jnp.inf, jnp.float32)
        zeros_f = jnp.zeros((16,), jnp.float32)
        zeros_i = jnp.zeros((16,), jnp.int32)
        ones_i = jnp.ones((16,), jnp.int32)

        def process_task(r, b):
            lo = r * _RV

            def ini(i, c):
                plsc.store_scatter(tbl, [i * 16 + lanes], minf)
                return c
            lax.fori_loop(0, _RV * _D // 16, ini, 0)

            def inic(i, c):
                plsc.store_scatter(cnts, [i * 16 + lanes], zeros_i)
                return c
            lax.fori_loop(0, _RV // 16, inic, 0)

            def do_chunk(c, carry):
                csz = jnp.where(c == _NCH - 1, _SCH_LAST, _SCH)
                hoff = b * _N + c * _SCH

                @pl.when(c == _NCH - 1)
                def _():
                    pltpu.sync_copy(idx_hbm.at[pl.ds(hoff, _SCH_LAST)],
                                    idxb.at[pl.ds(0, _SCH_LAST)])

                @pl.when(c < _NCH - 1)
                def _():
                    pltpu.sync_copy(idx_hbm.at[pl.ds(hoff, _SCH)], idxb)

                def scan_step(g, cnt):
                    v = plsc.load_gather(idxb, [g * 16 + lanes])
                    offs = v - lo
                    m = (offs >= 0) & (offs < _RV)
                    gidv = hoff + g * 16 + lanes
                    packed = offs * _PACK + gidv
                    ranks = plsc.cumsum(jnp.where(m, 1, 0))
                    plsc.store_scatter(selp, [cnt + ranks - 1], packed, mask=m)
                    return cnt + jnp.max(ranks)

                nv = csz // 16
                cnt = lax.fori_loop(0, nv, scan_step, 0)

                def do_sub(s, carry2):
                    k0 = s * _RB
                    valid = jnp.minimum(cnt - k0, _RB)

                    def unpack(u, c2):
                        v = plsc.load_gather(selp, [k0 + u * 16 + lanes])
                        g_ = jnp.minimum(v & (_PACK - 1), _B * _N - 1)
                        o_ = lax.shift_right_logical(v, 19)
                        plsc.store_scatter(gid, [u * 16 + lanes], g_)
                        plsc.store_scatter(goff, [u * 16 + lanes], o_)
                        mval = (u * 16 + lanes) < valid
                        plsc.addupdate_scatter(cnts, [o_], ones_i, mask=mval)
                        return c2
                    lax.fori_loop(0, _RB // 16, unpack, 0)

                    pltpu.async_copy(ft_hbm.at[gid], rows, gsem).wait()

                    def apply_pt(j, c3):
                        jv = lax.broadcast(j, (16,))
                        base = plsc.load_gather(goff, [jv]) * 128
                        for k in range(8):
                            kl = k * 16 + lanes
                            cur = plsc.load_gather(tbl, [base + kl])
                            val = plsc.load_gather(rows, [jv, kl])
                            plsc.store_scatter(tbl, [base + kl],
                                               jnp.maximum(cur, val))
                        return c3
                    lax.fori_loop(0, valid, apply_pt, 0)
                    return carry2

                nsb = lax.shift_right_logical(cnt + _RB - 1, 8)
                lax.fori_loop(0, nsb, do_sub, 0)
                return carry

            lax.fori_loop(0, _NCH, do_chunk, 0)

            def escan(e, ce):
                c16 = plsc.load_gather(cnts, [e * 16 + lanes])
                m = c16 == 0
                ranks = plsc.cumsum(jnp.where(m, 1, 0))
                plsc.store_scatter(selp, [ce + ranks - 1], e * 16 + lanes,
                                   mask=m)
                return ce + jnp.max(ranks)
            cnt_e = lax.fori_loop(0, _RV // 16, escan, 0)

            def ezero(j, cz):
                ov = plsc.load_gather(selp, [lax.broadcast(j, (16,))]) * 128
                for k in range(8):
                    plsc.store_scatter(tbl, [ov + k * 16 + lanes], zeros_f)
                return cz
            lax.fori_loop(0, cnt_e, ezero, 0)

            def cclip(e, cc):
                c16 = plsc.load_gather(cnts, [e * 16 + lanes])
                plsc.store_scatter(cnts, [e * 16 + lanes],
                                   jnp.maximum(c16, 1))
                return cc
            lax.fori_loop(0, _RV // 16, cclip, 0)

            pltpu.sync_copy(cnts, cnt_hbm.at[pl.ds(b * NUM_VOXELS + lo, _RV)])
            pltpu.sync_copy(
                tbl, vf_hbm.at[pl.ds((b * NUM_VOXELS + lo) * _D, _RV * _D)])

        for t in range(2):
            r = w + _NW * t

            def body(b, cb):
                process_task(r, b)
                return cb
            lax.fori_loop(0, _B, body, 0)

    return scat


_scatter_kernel = _make_scatter_kernel()


def kernel(features, xyz_coords_for_voxelization):
    B, D, N = features.shape
    idx = _idx_kernel(xyz_coords_for_voxelization.reshape(-1))
    ft = _transpose_features(features).reshape(B * N, D)
    vf_vm, counts = _scatter_kernel(ft, idx)
    vf = jnp.swapaxes(vf_vm.reshape(B, NUM_VOXELS, D), 1, 2)
    return (vf.reshape(B, D, GRID, GRID, GRID),
            idx.reshape(B, N),
            counts.reshape(B, 1, NUM_VOXELS))


# speedup vs baseline: 12.3751x; 12.3751x over previous
import functools

import jax
import jax.numpy as jnp
from jax import lax
from jax.experimental import pallas as pl
from jax.experimental.pallas import tpu as pltpu
from jax.experimental.pallas import tpu_sc as plsc

GRID = 32
NUM_VOXELS = GRID ** 3

_B, _D, _N = 4, 128, 100000
_NW = 32
_CH = 3136
_CH_LAST = _N - (_NW - 1) * _CH

_NR = 64
_RV = NUM_VOXELS // _NR
_SCH = 4096
_NCH = 25
_SCH_LAST = _N - (_NCH - 1) * _SCH
_RB = 128
_PACK = 1 << 19

_SC_PARAMS = pltpu.CompilerParams(needs_layout_passes=False)


def _make_idx_kernel():
    mesh = plsc.VectorSubcoreMesh(core_axis_name="c", subcore_axis_name="s")

    @functools.partial(
        pl.kernel,
        mesh=mesh,
        compiler_params=_SC_PARAMS,
        out_type=jax.ShapeDtypeStruct((_B * _N,), jnp.int32),
        scratch_types=[
            pltpu.VMEM((_CH * 3,), jnp.float32),
            pltpu.VMEM((_CH,), jnp.int32),
        ],
    )
    def idx_kernel(xyz_hbm, idx_hbm, xyz_v, out_v):
        w = lax.axis_index("s") * 2 + lax.axis_index("c")
        start = w * _CH
        lanes = lax.iota(jnp.int32, 16)
        nsteps = jnp.where(w == _NW - 1, _CH_LAST // 16, _CH // 16)

        for b in range(_B):
            @pl.when(w == _NW - 1)
            def _():
                pltpu.sync_copy(
                    xyz_hbm.at[pl.ds(b * _N * 3 + start * 3, _CH_LAST * 3)],
                    xyz_v.at[pl.ds(0, _CH_LAST * 3)])

            @pl.when(w < _NW - 1)
            def _():
                pltpu.sync_copy(
                    xyz_hbm.at[pl.ds(b * _N * 3 + start * 3, _CH * 3)], xyz_v)

            def step(g, carry):
                pos = (g * 16 + lanes) * 3
                x = plsc.load_gather(xyz_v, [pos])
                y = plsc.load_gather(xyz_v, [pos + 1])
                z = plsc.load_gather(xyz_v, [pos + 2])

                def vox(v):
                    vi = (v * float(GRID)).astype(jnp.int32)
                    return jnp.clip(vi, 0, GRID - 1)

                flat = vox(x) * (GRID * GRID) + vox(y) * GRID + vox(z)
                plsc.store_scatter(out_v, [g * 16 + lanes], flat)
                return carry

            lax.fori_loop(0, nsteps, step, 0)

            @pl.when(w == _NW - 1)
            def _():
                pltpu.sync_copy(out_v.at[pl.ds(0, _CH_LAST)],
                                idx_hbm.at[pl.ds(b * _N + start, _CH_LAST)])

            @pl.when(w < _NW - 1)
            def _():
                pltpu.sync_copy(out_v, idx_hbm.at[pl.ds(b * _N + start, _CH)])

    return idx_kernel


_idx_kernel = _make_idx_kernel()


_TRN = 2048


def _tr_body(f_ref, o_ref):
    o_ref[0] = jnp.swapaxes(f_ref[0], 0, 1)


def _transpose_features(features):
    nblk = (_N + _TRN - 1) // _TRN
    return pl.pallas_call(
        _tr_body,
        out_shape=jax.ShapeDtypeStruct((_B, _N, _D), jnp.float32),
        grid=(_B, nblk),
        in_specs=[pl.BlockSpec((1, _D, _TRN), lambda b, i: (b, 0, i))],
        out_specs=pl.BlockSpec((1, _TRN, _D), lambda b, i: (b, i, 0)),
    )(features)


def _make_scatter_kernel():
    mesh = plsc.VectorSubcoreMesh(core_axis_name="c", subcore_axis_name="s")

    @functools.partial(
        pl.kernel,
        mesh=mesh,
        compiler_params=_SC_PARAMS,
        out_type=(
            jax.ShapeDtypeStruct((_B * NUM_VOXELS * _D,), jnp.float32),
            jax.ShapeDtypeStruct((_B * NUM_VOXELS,), jnp.int32),
        ),
        scratch_types=[
            pltpu.VMEM((_RV * _D,), jnp.float32),
            pltpu.VMEM((2 * _SCH,), jnp.int32),
            pltpu.VMEM((_SCH,), jnp.int32),
            pltpu.VMEM((2 * _RB,), jnp.int32),
            pltpu.VMEM((2 * _RB,), jnp.int32),
            pltpu.VMEM((2, _RB, _D), jnp.float32),
            pltpu.VMEM((_RV,), jnp.int32),
            pltpu.SemaphoreType.DMA,
            pltpu.SemaphoreType.DMA,
        ],
    )
    def scat(ft_hbm, idx_hbm, vf_hbm, cnt_hbm,
             tbl, idxb, selp, gid, goff, rows, cnts, isem, gsem):
        w = lax.axis_index("s") * 2 + lax.axis_index("c")
        lanes = lax.iota(jnp.int32, 16)
        minf = jnp.full((16,), -jnp.inf, jnp.float32)
        zeros_f = jnp.zeros((16,), jnp.float32)
        zeros_i = jnp.zeros((16,), jnp.int32)
        ones_i = jnp.ones((16,), jnp.int32)

        def chunk_copy(b, c, par):
            hoff = b * _N + c * _SCH

            po = pl.multiple_of(par * _SCH, 8)

            @pl.when(c == _NCH - 1)
            def _():
                pltpu.async_copy(idx_hbm.at[pl.ds(hoff, _SCH_LAST)],
                                 idxb.at[pl.ds(po, _SCH_LAST)], isem)

            @pl.when(c < _NCH - 1)
            def _():
                pltpu.async_copy(idx_hbm.at[pl.ds(hoff, _SCH)],
                                 idxb.at[pl.ds(po, _SCH)], isem)

        def chunk_wait(b, c, par):
            hoff = b * _N + c * _SCH

            po = pl.multiple_of(par * _SCH, 8)

            @pl.when(c == _NCH - 1)
            def _():
                pltpu.make_async_copy(idx_hbm.at[pl.ds(hoff, _SCH_LAST)],
                                      idxb.at[pl.ds(po, _SCH_LAST)],
                                      isem).wait()

            @pl.when(c < _NCH - 1)
            def _():
                pltpu.make_async_copy(idx_hbm.at[pl.ds(hoff, _SCH)],
                                      idxb.at[pl.ds(po, _SCH)], isem).wait()

        def process_task(q, r, b, nxt_b, is_last):
            lo = r * _RV

            def ini(i, c):
                for k in range(8):
                    plsc.store_scatter(tbl, [(i * 8 + k) * 16 + lanes], minf)
                return c
            lax.fori_loop(0, _RV * _D // 128, ini, 0)

            def inic(i, c):
                plsc.store_scatter(cnts, [i * 16 + lanes], zeros_i)
                return c
            lax.fori_loop(0, _RV // 16, inic, 0)

            def do_chunk(c, carry):
                csz = jnp.where(c == _NCH - 1, _SCH_LAST, _SCH)
                par = (q + c) & 1
                parv = lax.broadcast(par, (16,))
                hoff = b * _N + c * _SCH

                chunk_copy(b, c, par)
                chunk_wait(b, c, par)

                nv = csz // 16

                def scan_step(g, cntv):
                    for kk in range(4):
                        g4 = g * 4 + kk
                        pos = g4 * 16 + lanes
                        v = plsc.load_gather(idxb, [parv * _SCH + pos])
                        offs = v - lo
                        m = (offs >= 0) & (offs < _RV)
                        m = m & lax.broadcast(g4 < nv, (16,))
                        packed = offs * _PACK + (hoff + pos)
                        ranks = plsc.cumsum(jnp.where(m, 1, 0))
                        plsc.store_scatter(selp, [cntv + ranks - 1], packed,
                                           mask=m)
                        cntv = cntv + plsc.all_reduce_population_count(m)
                    return cntv

                cntv = lax.fori_loop(0, (_SCH // 16) // 4, scan_step,
                                     jnp.zeros((16,), jnp.int32))
                cnt = jnp.max(cntv)

                def unpack(s, sp):
                    k0 = s * _RB
                    valid = jnp.minimum(cnt - k0, _RB)
                    spv = lax.broadcast(sp, (16,))

                    def u_step(u, c2):
                        ul = u * 16 + lanes
                        v = plsc.load_gather(selp, [k0 + ul])
                        g_ = jnp.minimum(v & (_PACK - 1), _B * _N - 1)
                        o_ = lax.shift_right_logical(v, 19)
                        plsc.store_scatter(gid, [spv * _RB + ul], g_)
                        plsc.store_scatter(goff, [spv * _RB + ul], o_)
                        plsc.addupdate_scatter(cnts, [o_], ones_i,
                                               mask=ul < valid)
                        return c2
                    lax.fori_loop(0, _RB // 16, u_step, 0)

                def gather_start(sp):
                    so = pl.multiple_of(sp * _RB, 8)
                    pltpu.async_copy(ft_hbm.at[gid.at[pl.ds(so, _RB)]],
                                     rows.at[sp], gsem)

                def gather_wait(sp):
                    so = pl.multiple_of(sp * _RB, 8)
                    pltpu.make_async_copy(ft_hbm.at[gid.at[pl.ds(so, _RB)]],
                                          rows.at[sp], gsem).wait()

                def do_sub(s, carry2):
                    sp = s & 1
                    k0 = s * _RB
                    valid = jnp.minimum(cnt - k0, _RB)
                    unpack(s, sp)
                    gather_start(sp)
                    gather_wait(sp)

                    spv = lax.broadcast(sp, (16,))

                    def apply_pt(j, c3):
                        jv = lax.broadcast(j, (16,))
                        base = plsc.load_gather(goff, [spv * _RB + jv]) * 128
                        for k in range(8):
                            kl = k * 16 + lanes
                            cur = plsc.load_gather(tbl, [base + kl])
                            val = plsc.load_gather(rows, [spv, jv, kl])
                            plsc.store_scatter(tbl, [base + kl],
                                               jnp.maximum(cur, val))
                        return c3
                    lax.fori_loop(0, valid, apply_pt, 0)
                    return carry2

                nsb = lax.shift_right_logical(cnt + _RB - 1, 7)
                lax.fori_loop(0, nsb, do_sub, 0)
                return carry

            lax.fori_loop(0, _NCH, do_chunk, 0)

            def escan(e, ce):
                c16 = plsc.load_gather(cnts, [e * 16 + lanes])
                m = c16 == 0
                ranks = plsc.cumsum(jnp.where(m, 1, 0))
                plsc.store_scatter(selp, [ce + ranks - 1], e * 16 + lanes,
                                   mask=m)
                return ce + jnp.max(ranks)
            cnt_e = lax.fori_loop(0, _RV // 16, escan, 0)

            def ezero(j, cz):
                ov = plsc.load_gather(selp, [lax.broadcast(j, (16,))]) * 128
                for k in range(8):
                    plsc.store_scatter(tbl, [ov + k * 16 + lanes], zeros_f)
                return cz
            lax.fori_loop(0, cnt_e, ezero, 0)

            def cclip(e, cc):
                c16 = plsc.load_gather(cnts, [e * 16 + lanes])
                plsc.store_scatter(cnts, [e * 16 + lanes],
                                   jnp.maximum(c16, 1))
                return cc
            lax.fori_loop(0, _RV // 16, cclip, 0)

            pltpu.sync_copy(cnts, cnt_hbm.at[pl.ds(b * NUM_VOXELS + lo, _RV)])
            pltpu.sync_copy(
                tbl, vf_hbm.at[pl.ds((b * NUM_VOXELS + lo) * _D, _RV * _D)])

        def task_body(q, cq):
            r = w + _NW * lax.shift_right_logical(q, 2)
            b = q & 3
            process_task(q, r, b, (q + 1) & 3, q == 2 * _B - 1)
            return cq
        lax.fori_loop(0, 2 * _B, task_body, 0)

    return scat


_scatter_kernel = _make_scatter_kernel()


def kernel(features, xyz_coords_for_voxelization):
    B, D, N = features.shape
    idx = _idx_kernel(xyz_coords_for_voxelization.reshape(-1))
    ft = _transpose_features(features).reshape(B * N, D)
    vf_vm, counts = _scatter_kernel(ft, idx)
    vf = jnp.swapaxes(vf_vm.reshape(B, NUM_VOXELS, D), 1, 2)
    return (vf.reshape(B, D, GRID, GRID, GRID),
            idx.reshape(B, N),
            counts.reshape(B, 1, NUM_VOXELS))


# speedup vs baseline: 12.3837x; 1.0007x over previous
import functools

import jax
import jax.numpy as jnp
from jax import lax
from jax.experimental import pallas as pl
from jax.experimental.pallas import tpu as pltpu
from jax.experimental.pallas import tpu_sc as plsc

GRID = 32
NUM_VOXELS = GRID ** 3

_B, _D, _N = 4, 128, 100000
_NW = 32
_CH = 3136
_CH_LAST = _N - (_NW - 1) * _CH

_NR = 64
_RV = NUM_VOXELS // _NR
_SCH = 4096
_NCH = 25
_SCH_LAST = _N - (_NCH - 1) * _SCH
_RB = 128
_PACK = 1 << 19

_SC_PARAMS = pltpu.CompilerParams(needs_layout_passes=False)


def _make_idx_kernel():
    mesh = plsc.VectorSubcoreMesh(core_axis_name="c", subcore_axis_name="s")

    @functools.partial(
        pl.kernel,
        mesh=mesh,
        compiler_params=_SC_PARAMS,
        out_type=jax.ShapeDtypeStruct((_B * _N,), jnp.int32),
        scratch_types=[
            pltpu.VMEM((_CH * 3,), jnp.float32),
            pltpu.VMEM((_CH,), jnp.int32),
        ],
    )
    def idx_kernel(xyz_hbm, idx_hbm, xyz_v, out_v):
        w = lax.axis_index("s") * 2 + lax.axis_index("c")
        start = w * _CH
        lanes = lax.iota(jnp.int32, 16)
        nsteps = jnp.where(w == _NW - 1, _CH_LAST // 16, _CH // 16)

        for b in range(_B):
            @pl.when(w == _NW - 1)
            def _():
                pltpu.sync_copy(
                    xyz_hbm.at[pl.ds(b * _N * 3 + start * 3, _CH_LAST * 3)],
                    xyz_v.at[pl.ds(0, _CH_LAST * 3)])

            @pl.when(w < _NW - 1)
            def _():
                pltpu.sync_copy(
                    xyz_hbm.at[pl.ds(b * _N * 3 + start * 3, _CH * 3)], xyz_v)

            def step(g, carry):
                pos = (g * 16 + lanes) * 3
                x = plsc.load_gather(xyz_v, [pos])
                y = plsc.load_gather(xyz_v, [pos + 1])
                z = plsc.load_gather(xyz_v, [pos + 2])

                def vox(v):
                    vi = (v * float(GRID)).astype(jnp.int32)
                    return jnp.clip(vi, 0, GRID - 1)

                flat = vox(x) * (GRID * GRID) + vox(y) * GRID + vox(z)
                plsc.store_scatter(out_v, [g * 16 + lanes], flat)
                return carry

            lax.fori_loop(0, nsteps, step, 0)

            @pl.when(w == _NW - 1)
            def _():
                pltpu.sync_copy(out_v.at[pl.ds(0, _CH_LAST)],
                                idx_hbm.at[pl.ds(b * _N + start, _CH_LAST)])

            @pl.when(w < _NW - 1)
            def _():
                pltpu.sync_copy(out_v, idx_hbm.at[pl.ds(b * _N + start, _CH)])

    return idx_kernel


_idx_kernel = _make_idx_kernel()


_TRN = 2048


def _tr_body(f_ref, o_ref):
    o_ref[0] = jnp.swapaxes(f_ref[0], 0, 1)


def _transpose_features(features):
    nblk = (_N + _TRN - 1) // _TRN
    return pl.pallas_call(
        _tr_body,
        out_shape=jax.ShapeDtypeStruct((_B, _N, _D), jnp.float32),
        grid=(_B, nblk),
        in_specs=[pl.BlockSpec((1, _D, _TRN), lambda b, i: (b, 0, i))],
        out_specs=pl.BlockSpec((1, _TRN, _D), lambda b, i: (b, i, 0)),
    )(features)


def _make_scatter_kernel():
    mesh = plsc.VectorSubcoreMesh(core_axis_name="c", subcore_axis_name="s")

    @functools.partial(
        pl.kernel,
        mesh=mesh,
        compiler_params=_SC_PARAMS,
        out_type=(
            jax.ShapeDtypeStruct((_B * NUM_VOXELS * _D,), jnp.float32),
            jax.ShapeDtypeStruct((_B * NUM_VOXELS,), jnp.int32),
        ),
        scratch_types=[
            pltpu.VMEM((_RV * _D,), jnp.float32),
            pltpu.VMEM((2 * _SCH,), jnp.int32),
            pltpu.VMEM((_SCH,), jnp.int32),
            pltpu.VMEM((2 * _RB,), jnp.int32),
            pltpu.VMEM((2 * _RB,), jnp.int32),
            pltpu.VMEM((2, _RB, _D), jnp.float32),
            pltpu.VMEM((_RV,), jnp.int32),
            pltpu.SemaphoreType.DMA,
            pltpu.SemaphoreType.DMA,
        ],
    )
    def scat(ft_hbm, idx_hbm, vf_hbm, cnt_hbm,
             tbl, idxb, selp, gid, goff, rows, cnts, isem, gsem):
        w = lax.axis_index("s") * 2 + lax.axis_index("c")
        lanes = lax.iota(jnp.int32, 16)
        minf = jnp.full((16,), -jnp.inf, jnp.float32)
        zeros_f = jnp.zeros((16,), jnp.float32)
        zeros_i = jnp.zeros((16,), jnp.int32)
        ones_i = jnp.ones((16,), jnp.int32)

        def chunk_copy(b, c, par):
            hoff = b * _N + c * _SCH

            po = pl.multiple_of(par * _SCH, 8)

            @pl.when(c == _NCH - 1)
            def _():
                pltpu.async_copy(idx_hbm.at[pl.ds(hoff, _SCH_LAST)],
                                 idxb.at[pl.ds(po, _SCH_LAST)], isem)

            @pl.when(c < _NCH - 1)
            def _():
                pltpu.async_copy(idx_hbm.at[pl.ds(hoff, _SCH)],
                                 idxb.at[pl.ds(po, _SCH)], isem)

        def chunk_wait(b, c, par):
            hoff = b * _N + c * _SCH

            po = pl.multiple_of(par * _SCH, 8)

            @pl.when(c == _NCH - 1)
            def _():
                pltpu.make_async_copy(idx_hbm.at[pl.ds(hoff, _SCH_LAST)],
                                      idxb.at[pl.ds(po, _SCH_LAST)],
                                      isem).wait()

            @pl.when(c < _NCH - 1)
            def _():
                pltpu.make_async_copy(idx_hbm.at[pl.ds(hoff, _SCH)],
                                      idxb.at[pl.ds(po, _SCH)], isem).wait()

        def process_task(q, r, b, nxt_b, is_last):
            lo = r * _RV

            def ini(i, c):
                for k in range(8):
                    plsc.store_scatter(tbl, [(i * 8 + k) * 16 + lanes], minf)
                return c
            lax.fori_loop(0, _RV * _D // 128, ini, 0)

            def inic(i, c):
                plsc.store_scatter(cnts, [i * 16 + lanes], zeros_i)
                return c
            lax.fori_loop(0, _RV // 16, inic, 0)

            def do_chunk(c, carry):
                csz = jnp.where(c == _NCH - 1, _SCH_LAST, _SCH)
                par = c & 1
                parv = lax.broadcast(par, (16,))
                hoff = b * _N + c * _SCH

                chunk_copy(b, c, par)
                chunk_wait(b, c, par)

                nv = csz // 16

                def scan_step(g, cntv):
                    for kk in range(4):
                        g4 = g * 4 + kk
                        pos = g4 * 16 + lanes
                        v = plsc.load_gather(idxb, [parv * _SCH + pos])
                        offs = v - lo
                        m = (offs >= 0) & (offs < _RV)
                        m = m & lax.broadcast(g4 < nv, (16,))
                        packed = offs * _PACK + (hoff + pos)
                        ranks = plsc.cumsum(jnp.where(m, 1, 0))
                        plsc.store_scatter(selp, [cntv + ranks - 1], packed,
                                           mask=m)
                        cntv = cntv + plsc.all_reduce_population_count(m)
                    return cntv

                cntv = lax.fori_loop(0, (_SCH // 16) // 4, scan_step,
                                     jnp.zeros((16,), jnp.int32))
                cnt = jnp.max(cntv)

                def unpack(s, sp):
                    k0 = s * _RB
                    valid = jnp.minimum(cnt - k0, _RB)
                    spv = lax.broadcast(sp, (16,))

                    def u_step(u, c2):
                        ul = u * 16 + lanes
                        v = plsc.load_gather(selp, [k0 + ul])
                        g_ = jnp.minimum(v & (_PACK - 1), _B * _N - 1)
                        o_ = lax.shift_right_logical(v, 19)
                        plsc.store_scatter(gid, [spv * _RB + ul], g_)
                        plsc.store_scatter(goff, [spv * _RB + ul], o_)
                        plsc.addupdate_scatter(cnts, [o_], ones_i,
                                               mask=ul < valid)
                        return c2
                    lax.fori_loop(0, _RB // 16, u_step, 0)

                def gather_start(sp):
                    so = pl.multiple_of(sp * _RB, 8)
                    pltpu.async_copy(ft_hbm.at[gid.at[pl.ds(so, _RB)]],
                                     rows.at[sp], gsem)

                def gather_wait(sp):
                    so = pl.multiple_of(sp * _RB, 8)
                    pltpu.make_async_copy(ft_hbm.at[gid.at[pl.ds(so, _RB)]],
                                          rows.at[sp], gsem).wait()

                def do_sub(s, carry2):
                    sp = s & 1
                    k0 = s * _RB
                    valid = jnp.minimum(cnt - k0, _RB)
                    gather_wait(sp)

                    @pl.when(s + 1 < nsb)
                    def _():
                        unpack(s + 1, 1 - sp)
                        gather_start(1 - sp)

                    spv = lax.broadcast(sp, (16,))

                    def apply_pt(j, c3):
                        jv = lax.broadcast(j, (16,))
                        base = plsc.load_gather(goff, [spv * _RB + jv]) * 128
                        for k in range(8):
                            kl = k * 16 + lanes
                            cur = plsc.load_gather(tbl, [base + kl])
                            val = plsc.load_gather(rows, [spv, jv, kl])
                            plsc.store_scatter(tbl, [base + kl],
                                               jnp.maximum(cur, val))
                        return c3
                    lax.fori_loop(0, valid, apply_pt, 0)
                    return carry2

                nsb = lax.shift_right_logical(cnt + _RB - 1, 7)

                @pl.when(nsb > 0)
                def _():
                    unpack(0, 0)
                    gather_start(0)

                lax.fori_loop(0, nsb, do_sub, 0)
                return carry

            lax.fori_loop(0, _NCH, do_chunk, 0)

            def escan(e, ce):
                c16 = plsc.load_gather(cnts, [e * 16 + lanes])
                m = c16 == 0
                ranks = plsc.cumsum(jnp.where(m, 1, 0))
                plsc.store_scatter(selp, [ce + ranks - 1], e * 16 + lanes,
                                   mask=m)
                return ce + jnp.max(ranks)
            cnt_e = lax.fori_loop(0, _RV // 16, escan, 0)

            def ezero(j, cz):
                ov = plsc.load_gather(selp, [lax.broadcast(j, (16,))]) * 128
                for k in range(8):
                    plsc.store_scatter(tbl, [ov + k * 16 + lanes], zeros_f)
                return cz
            lax.fori_loop(0, cnt_e, ezero, 0)

            def cclip(e, cc):
                c16 = plsc.load_gather(cnts, [e * 16 + lanes])
                plsc.store_scatter(cnts, [e * 16 + lanes],
                                   jnp.maximum(c16, 1))
                return cc
            lax.fori_loop(0, _RV // 16, cclip, 0)

            pltpu.sync_copy(cnts, cnt_hbm.at[pl.ds(b * NUM_VOXELS + lo, _RV)])
            pltpu.sync_copy(
                tbl, vf_hbm.at[pl.ds((b * NUM_VOXELS + lo) * _D, _RV * _D)])

        def task_body(q, cq):
            r = w + _NW * lax.shift_right_logical(q, 2)
            b = q & 3
            process_task(q, r, b, (q + 1) & 3, q == 2 * _B - 1)
            return cq
        lax.fori_loop(0, 2 * _B, task_body, 0)

    return scat


_scatter_kernel = _make_scatter_kernel()


def kernel(features, xyz_coords_for_voxelization):
    B, D, N = features.shape
    idx = _idx_kernel(xyz_coords_for_voxelization.reshape(-1))
    ft = _transpose_features(features).reshape(B * N, D)
    vf_vm, counts = _scatter_kernel(ft, idx)
    vf = jnp.swapaxes(vf_vm.reshape(B, NUM_VOXELS, D), 1, 2)
    return (vf.reshape(B, D, GRID, GRID, GRID),
            idx.reshape(B, N),
            counts.reshape(B, 1, NUM_VOXELS))


# speedup vs baseline: 12.9374x; 1.0447x over previous
import functools

import jax
import jax.numpy as jnp
from jax import lax
from jax.experimental import pallas as pl
from jax.experimental.pallas import tpu as pltpu
from jax.experimental.pallas import tpu_sc as plsc

GRID = 32
NUM_VOXELS = GRID ** 3

_B, _D, _N = 4, 128, 100000
_NW = 32
_CH = 3136
_CH_LAST = _N - (_NW - 1) * _CH

_NR = 64
_RV = NUM_VOXELS // _NR
_SCH = 4096
_NCH = 25
_SCH_LAST = _N - (_NCH - 1) * _SCH
_RB = 128
_PACK = 1 << 19

_SC_PARAMS = pltpu.CompilerParams(needs_layout_passes=False)


def _make_idx_kernel():
    mesh = plsc.VectorSubcoreMesh(core_axis_name="c", subcore_axis_name="s")

    @functools.partial(
        pl.kernel,
        mesh=mesh,
        compiler_params=_SC_PARAMS,
        out_type=jax.ShapeDtypeStruct((_B * _N,), jnp.int32),
        scratch_types=[
            pltpu.VMEM((_CH * 3,), jnp.float32),
            pltpu.VMEM((_CH,), jnp.int32),
        ],
    )
    def idx_kernel(xyz_hbm, idx_hbm, xyz_v, out_v):
        w = lax.axis_index("s") * 2 + lax.axis_index("c")
        start = w * _CH
        lanes = lax.iota(jnp.int32, 16)
        nsteps = jnp.where(w == _NW - 1, _CH_LAST // 16, _CH // 16)

        for b in range(_B):
            @pl.when(w == _NW - 1)
            def _():
                pltpu.sync_copy(
                    xyz_hbm.at[pl.ds(b * _N * 3 + start * 3, _CH_LAST * 3)],
                    xyz_v.at[pl.ds(0, _CH_LAST * 3)])

            @pl.when(w < _NW - 1)
            def _():
                pltpu.sync_copy(
                    xyz_hbm.at[pl.ds(b * _N * 3 + start * 3, _CH * 3)], xyz_v)

            def step(g, carry):
                pos = (g * 16 + lanes) * 3
                x = plsc.load_gather(xyz_v, [pos])
                y = plsc.load_gather(xyz_v, [pos + 1])
                z = plsc.load_gather(xyz_v, [pos + 2])

                def vox(v):
                    vi = (v * float(GRID)).astype(jnp.int32)
                    return jnp.clip(vi, 0, GRID - 1)

                flat = vox(x) * (GRID * GRID) + vox(y) * GRID + vox(z)
                plsc.store_scatter(out_v, [g * 16 + lanes], flat)
                return carry

            lax.fori_loop(0, nsteps, step, 0)

            @pl.when(w == _NW - 1)
            def _():
                pltpu.sync_copy(out_v.at[pl.ds(0, _CH_LAST)],
                                idx_hbm.at[pl.ds(b * _N + start, _CH_LAST)])

            @pl.when(w < _NW - 1)
            def _():
                pltpu.sync_copy(out_v, idx_hbm.at[pl.ds(b * _N + start, _CH)])

    return idx_kernel


_idx_kernel = _make_idx_kernel()


_TRN = 2048


def _tr_body(f_ref, o_ref):
    o_ref[0] = jnp.swapaxes(f_ref[0], 0, 1)


def _transpose_features(features):
    nblk = (_N + _TRN - 1) // _TRN
    return pl.pallas_call(
        _tr_body,
        out_shape=jax.ShapeDtypeStruct((_B, _N, _D), jnp.float32),
        grid=(_B, nblk),
        in_specs=[pl.BlockSpec((1, _D, _TRN), lambda b, i: (b, 0, i))],
        out_specs=pl.BlockSpec((1, _TRN, _D), lambda b, i: (b, i, 0)),
    )(features)


def _make_scatter_kernel():
    mesh = plsc.VectorSubcoreMesh(core_axis_name="c", subcore_axis_name="s")

    @functools.partial(
        pl.kernel,
        mesh=mesh,
        compiler_params=_SC_PARAMS,
        out_type=(
            jax.ShapeDtypeStruct((_B * NUM_VOXELS * _D,), jnp.float32),
            jax.ShapeDtypeStruct((_B * NUM_VOXELS,), jnp.int32),
        ),
        scratch_types=[
            pltpu.VMEM((_RV * _D,), jnp.float32),
            pltpu.VMEM((2 * _SCH,), jnp.int32),
            pltpu.VMEM((_SCH,), jnp.int32),
            pltpu.VMEM((2 * _RB,), jnp.int32),
            pltpu.VMEM((2 * _RB,), jnp.int32),
            pltpu.VMEM((2, _RB, _D), jnp.float32),
            pltpu.VMEM((_RV,), jnp.int32),
            pltpu.SemaphoreType.DMA,
            pltpu.SemaphoreType.DMA,
        ],
    )
    def scat(ft_hbm, idx_hbm, vf_hbm, cnt_hbm,
             tbl, idxb, selp, gid, goff, rows, cnts, isem, gsem):
        w = lax.axis_index("s") * 2 + lax.axis_index("c")
        lanes = lax.iota(jnp.int32, 16)
        minf = jnp.full((16,), -jnp.inf, jnp.float32)
        zeros_f = jnp.zeros((16,), jnp.float32)
        zeros_i = jnp.zeros((16,), jnp.int32)
        ones_i = jnp.ones((16,), jnp.int32)

        def chunk_copy(b, c, par):
            hoff = b * _N + c * _SCH

            po = pl.multiple_of(par * _SCH, 8)

            @pl.when(c == _NCH - 1)
            def _():
                pltpu.async_copy(idx_hbm.at[pl.ds(hoff, _SCH_LAST)],
                                 idxb.at[pl.ds(po, _SCH_LAST)], isem)

            @pl.when(c < _NCH - 1)
            def _():
                pltpu.async_copy(idx_hbm.at[pl.ds(hoff, _SCH)],
                                 idxb.at[pl.ds(po, _SCH)], isem)

        def chunk_wait(b, c, par):
            hoff = b * _N + c * _SCH

            po = pl.multiple_of(par * _SCH, 8)

            @pl.when(c == _NCH - 1)
            def _():
                pltpu.make_async_copy(idx_hbm.at[pl.ds(hoff, _SCH_LAST)],
                                      idxb.at[pl.ds(po, _SCH_LAST)],
                                      isem).wait()

            @pl.when(c < _NCH - 1)
            def _():
                pltpu.make_async_copy(idx_hbm.at[pl.ds(hoff, _SCH)],
                                      idxb.at[pl.ds(po, _SCH)], isem).wait()

        def process_task(q, r, b, nxt_b, is_last):
            lo = r * _RV

            def ini(i, c):
                for k in range(8):
                    plsc.store_scatter(tbl, [(i * 8 + k) * 16 + lanes], minf)
                return c
            with jax.named_scope("init"):
                lax.fori_loop(0, _RV * _D // 128, ini, 0)

            def inic(i, c):
                plsc.store_scatter(cnts, [i * 16 + lanes], zeros_i)
                return c
            lax.fori_loop(0, _RV // 16, inic, 0)

            def do_chunk(c, carry):
                csz = jnp.where(c == _NCH - 1, _SCH_LAST, _SCH)
                par = c & 1
                parv = lax.broadcast(par, (16,))
                hoff = b * _N + c * _SCH

                with jax.named_scope("idx_dma"):
                    chunk_copy(b, c, par)
                    chunk_wait(b, c, par)

                nv = csz // 16

                def scan_step(g, cntv):
                    for kk in range(4):
                        g4 = g * 4 + kk
                        pos = g4 * 16 + lanes
                        v = plsc.load_gather(idxb, [parv * _SCH + pos])
                        offs = v - lo
                        m = (offs >= 0) & (offs < _RV)
                        m = m & lax.broadcast(g4 < nv, (16,))
                        packed = offs * _PACK + (hoff + pos)
                        ranks = plsc.cumsum(jnp.where(m, 1, 0))
                        plsc.store_scatter(selp, [cntv + ranks - 1], packed,
                                           mask=m)
                        cntv = cntv + plsc.all_reduce_population_count(m)
                    return cntv

                with jax.named_scope("scan"):
                    cntv = lax.fori_loop(0, (_SCH // 16) // 4, scan_step,
                                         jnp.zeros((16,), jnp.int32))
                    cnt = jnp.max(cntv)

                def unpack(s, sp):
                    k0 = s * _RB
                    valid = jnp.minimum(cnt - k0, _RB)
                    spv = lax.broadcast(sp, (16,))

                    def u_step(u, c2):
                        ul = u * 16 + lanes
                        v = plsc.load_gather(selp, [k0 + ul])
                        g_ = jnp.minimum(v & (_PACK - 1), _B * _N - 1)
                        o_ = lax.shift_right_logical(v, 19)
                        plsc.store_scatter(gid, [spv * _RB + ul], g_)
                        plsc.store_scatter(goff, [spv * _RB + ul], o_)
                        plsc.addupdate_scatter(cnts, [o_], ones_i,
                                               mask=ul < valid)
                        return c2
                    lax.fori_loop(0, _RB // 16, u_step, 0)

                def gather_start(sp):
                    so = pl.multiple_of(sp * _RB, 8)
                    pltpu.async_copy(ft_hbm.at[gid.at[pl.ds(so, _RB)]],
                                     rows.at[sp], gsem)

                def gather_wait(sp):
                    so = pl.multiple_of(sp * _RB, 8)
                    pltpu.make_async_copy(ft_hbm.at[gid.at[pl.ds(so, _RB)]],
                                          rows.at[sp], gsem).wait()

                def do_sub(s, carry2):
                    sp = s & 1
                    k0 = s * _RB
                    valid = jnp.minimum(cnt - k0, _RB)
                    gather_wait(sp)

                    @pl.when(s + 1 < nsb)
                    def _():
                        unpack(s + 1, 1 - sp)
                        gather_start(1 - sp)

                    spv = lax.broadcast(sp, (16,))

                    def apply_pt(j, c3):
                        jv = lax.broadcast(j, (16,))
                        base = plsc.load_gather(goff, [spv * _RB + jv]) * 128
                        for k in range(8):
                            kl = k * 16 + lanes
                            cur = plsc.load_gather(tbl, [base + kl])
                            val = plsc.load_gather(rows, [spv, jv, kl])
                            plsc.store_scatter(tbl, [base + kl],
                                               jnp.maximum(cur, val))
                        return c3
                    lax.fori_loop(0, valid, apply_pt, 0)
                    return carry2

                nsb = lax.shift_right_logical(cnt + _RB - 1, 7)

                with jax.named_scope("apply"):
                    @pl.when(nsb > 0)
                    def _():
                        unpack(0, 0)
                        gather_start(0)

                    lax.fori_loop(0, nsb, do_sub, 0)
                return carry

            lax.fori_loop(0, _NCH, do_chunk, 0)

            def escan(e, ce):
                c16 = plsc.load_gather(cnts, [e * 16 + lanes])
                m = c16 == 0
                ranks = plsc.cumsum(jnp.where(m, 1, 0))
                plsc.store_scatter(selp, [ce + ranks - 1], e * 16 + lanes,
                                   mask=m)
                return ce + jnp.max(ranks)
            cnt_e = lax.fori_loop(0, _RV // 16, escan, 0)

            def ezero(j, cz):
                ov = plsc.load_gather(selp, [lax.broadcast(j, (16,))]) * 128
                for k in range(8):
                    plsc.store_scatter(tbl, [ov + k * 16 + lanes], zeros_f)
                return cz
            lax.fori_loop(0, cnt_e, ezero, 0)

            def cclip(e, cc):
                c16 = plsc.load_gather(cnts, [e * 16 + lanes])
                plsc.store_scatter(cnts, [e * 16 + lanes],
                                   jnp.maximum(c16, 1))
                return cc
            lax.fori_loop(0, _RV // 16, cclip, 0)

            pltpu.sync_copy(cnts, cnt_hbm.at[pl.ds(b * NUM_VOXELS + lo, _RV)])
            pltpu.sync_copy(
                tbl, vf_hbm.at[pl.ds((b * NUM_VOXELS + lo) * _D, _RV * _D)])

        def task_body(q, cq):
            r = w + _NW * lax.shift_right_logical(q, 2)
            b = q & 3
            process_task(q, r, b, (q + 1) & 3, q == 2 * _B - 1)
            return cq
        lax.fori_loop(0, 2 * _B, task_body, 0)

    return scat


_scatter_kernel = _make_scatter_kernel()


def kernel(features, xyz_coords_for_voxelization):
    B, D, N = features.shape
    idx = _idx_kernel(xyz_coords_for_voxelization.reshape(-1))
    ft = _transpose_features(features).reshape(B * N, D)
    vf_vm, counts = _scatter_kernel(ft, idx)
    vf = jnp.swapaxes(vf_vm.reshape(B, NUM_VOXELS, D), 1, 2)
    return (vf.reshape(B, D, GRID, GRID, GRID),
            idx.reshape(B, N),
            counts.reshape(B, 1, NUM_VOXELS))


# speedup vs baseline: 17.4551x; 1.3492x over previous
import functools

import jax
import jax.numpy as jnp
from jax import lax
from jax.experimental import pallas as pl
from jax.experimental.pallas import tpu as pltpu
from jax.experimental.pallas import tpu_sc as plsc

GRID = 32
NUM_VOXELS = GRID ** 3

_B, _D, _N = 4, 128, 100000
_NW = 32
_CH = 3136
_CH_LAST = _N - (_NW - 1) * _CH

_NR = 64
_RV = NUM_VOXELS // _NR
_SCH = 8192
_NCH = 13
_SCH_LAST = _N - (_NCH - 1) * _SCH
_RB = 128
_PACK = 1 << 19

_SC_PARAMS = pltpu.CompilerParams(needs_layout_passes=False)


def _make_idx_kernel():
    mesh = plsc.VectorSubcoreMesh(core_axis_name="c", subcore_axis_name="s")

    @functools.partial(
        pl.kernel,
        mesh=mesh,
        compiler_params=_SC_PARAMS,
        out_type=jax.ShapeDtypeStruct((_B * _N,), jnp.int32),
        scratch_types=[
            pltpu.VMEM((_CH * 3,), jnp.float32),
            pltpu.VMEM((_CH,), jnp.int32),
        ],
    )
    def idx_kernel(xyz_hbm, idx_hbm, xyz_v, out_v):
        w = lax.axis_index("s") * 2 + lax.axis_index("c")
        start = w * _CH
        lanes = lax.iota(jnp.int32, 16)
        nsteps = jnp.where(w == _NW - 1, _CH_LAST // 16, _CH // 16)

        for b in range(_B):
            @pl.when(w == _NW - 1)
            def _():
                pltpu.sync_copy(
                    xyz_hbm.at[pl.ds(b * _N * 3 + start * 3, _CH_LAST * 3)],
                    xyz_v.at[pl.ds(0, _CH_LAST * 3)])

            @pl.when(w < _NW - 1)
            def _():
                pltpu.sync_copy(
                    xyz_hbm.at[pl.ds(b * _N * 3 + start * 3, _CH * 3)], xyz_v)

            def step(g, carry):
                pos = (g * 16 + lanes) * 3
                x = plsc.load_gather(xyz_v, [pos])
                y = plsc.load_gather(xyz_v, [pos + 1])
                z = plsc.load_gather(xyz_v, [pos + 2])

                def vox(v):
                    vi = (v * float(GRID)).astype(jnp.int32)
                    return jnp.clip(vi, 0, GRID - 1)

                flat = vox(x) * (GRID * GRID) + vox(y) * GRID + vox(z)
                plsc.store_scatter(out_v, [g * 16 + lanes], flat)
                return carry

            lax.fori_loop(0, nsteps, step, 0)

            @pl.when(w == _NW - 1)
            def _():
                pltpu.sync_copy(out_v.at[pl.ds(0, _CH_LAST)],
                                idx_hbm.at[pl.ds(b * _N + start, _CH_LAST)])

            @pl.when(w < _NW - 1)
            def _():
                pltpu.sync_copy(out_v, idx_hbm.at[pl.ds(b * _N + start, _CH)])

    return idx_kernel


_idx_kernel = _make_idx_kernel()


_TRN = 2048


def _tr_body(f_ref, o_ref):
    o_ref[0] = jnp.swapaxes(f_ref[0], 0, 1)


def _transpose_features(features):
    nblk = (_N + _TRN - 1) // _TRN
    return pl.pallas_call(
        _tr_body,
        out_shape=jax.ShapeDtypeStruct((_B, _N, _D), jnp.float32),
        grid=(_B, nblk),
        in_specs=[pl.BlockSpec((1, _D, _TRN), lambda b, i: (b, 0, i))],
        out_specs=pl.BlockSpec((1, _TRN, _D), lambda b, i: (b, i, 0)),
    )(features)


def _make_scatter_kernel():
    mesh = plsc.VectorSubcoreMesh(core_axis_name="c", subcore_axis_name="s")

    @functools.partial(
        pl.kernel,
        mesh=mesh,
        compiler_params=_SC_PARAMS,
        out_type=(
            jax.ShapeDtypeStruct((_B * NUM_VOXELS * _D,), jnp.float32),
            jax.ShapeDtypeStruct((_B * NUM_VOXELS,), jnp.int32),
        ),
        scratch_types=[
            pltpu.VMEM((_RV * _D,), jnp.float32),
            pltpu.VMEM((2 * _SCH,), jnp.int32),
            pltpu.VMEM((_SCH,), jnp.int32),
            pltpu.VMEM((2 * _RB,), jnp.int32),
            pltpu.VMEM((2 * _RB,), jnp.int32),
            pltpu.VMEM((2, _RB, _D), jnp.float32),
            pltpu.VMEM((_RV,), jnp.int32),
            pltpu.SemaphoreType.DMA,
            pltpu.SemaphoreType.DMA,
        ],
    )
    def scat(ft_hbm, idx_hbm, vf_hbm, cnt_hbm,
             tbl, idxb, selp, gid, goff, rows, cnts, isem, gsem):
        w = lax.axis_index("s") * 2 + lax.axis_index("c")
        lanes = lax.iota(jnp.int32, 16)
        minf = jnp.full((16,), -jnp.inf, jnp.float32)
        zeros_f = jnp.zeros((16,), jnp.float32)
        zeros_i = jnp.zeros((16,), jnp.int32)
        ones_i = jnp.ones((16,), jnp.int32)

        def chunk_copy(b, c, par):
            hoff = b * _N + c * _SCH

            po = pl.multiple_of(par * _SCH, 8)

            @pl.when(c == _NCH - 1)
            def _():
                pltpu.async_copy(idx_hbm.at[pl.ds(hoff, _SCH_LAST)],
                                 idxb.at[pl.ds(po, _SCH_LAST)], isem)

            @pl.when(c < _NCH - 1)
            def _():
                pltpu.async_copy(idx_hbm.at[pl.ds(hoff, _SCH)],
                                 idxb.at[pl.ds(po, _SCH)], isem)

        def chunk_wait(b, c, par):
            hoff = b * _N + c * _SCH

            po = pl.multiple_of(par * _SCH, 8)

            @pl.when(c == _NCH - 1)
            def _():
                pltpu.make_async_copy(idx_hbm.at[pl.ds(hoff, _SCH_LAST)],
                                      idxb.at[pl.ds(po, _SCH_LAST)],
                                      isem).wait()

            @pl.when(c < _NCH - 1)
            def _():
                pltpu.make_async_copy(idx_hbm.at[pl.ds(hoff, _SCH)],
                                      idxb.at[pl.ds(po, _SCH)], isem).wait()

        def process_task(q, r, b, nxt_b, is_last):
            lo = r * _RV

            def ini(i, c):
                for k in range(8):
                    plsc.store_scatter(tbl, [(i * 8 + k) * 16 + lanes], minf)
                return c
            with jax.named_scope("init"):
                lax.fori_loop(0, _RV * _D // 128, ini, 0)

            def inic(i, c):
                plsc.store_scatter(cnts, [i * 16 + lanes], zeros_i)
                return c
            lax.fori_loop(0, _RV // 16, inic, 0)

            def do_chunk(c, carry):
                csz = jnp.where(c == _NCH - 1, _SCH_LAST, _SCH)
                par = c & 1
                parv = lax.broadcast(par, (16,))
                hoff = b * _N + c * _SCH

                with jax.named_scope("idx_dma"):
                    chunk_copy(b, c, par)
                    chunk_wait(b, c, par)

                nv = csz // 16

                def scan_step(g, cntv):
                    for kk in range(4):
                        g4 = g * 4 + kk
                        pos = g4 * 16 + lanes
                        v = plsc.load_gather(idxb, [parv * _SCH + pos])
                        offs = v - lo
                        m = (offs >= 0) & (offs < _RV)
                        m = m & lax.broadcast(g4 < nv, (16,))
                        packed = offs * _PACK + (hoff + pos)
                        ranks = plsc.cumsum(jnp.where(m, 1, 0))
                        plsc.store_scatter(selp, [cntv + ranks - 1], packed,
                                           mask=m)
                        cntv = cntv + plsc.all_reduce_population_count(m)
                    return cntv

                with jax.named_scope("scan"):
                    cntv = lax.fori_loop(0, (_SCH // 16) // 4, scan_step,
                                         jnp.zeros((16,), jnp.int32))
                    cnt = jnp.max(cntv)

                def unpack(s, sp):
                    k0 = s * _RB
                    valid = jnp.minimum(cnt - k0, _RB)
                    spv = lax.broadcast(sp, (16,))

                    def u_step(u, c2):
                        ul = u * 16 + lanes
                        v = plsc.load_gather(selp, [k0 + ul])
                        g_ = jnp.minimum(v & (_PACK - 1), _B * _N - 1)
                        o_ = lax.shift_right_logical(v, 19)
                        plsc.store_scatter(gid, [spv * _RB + ul], g_)
                        plsc.store_scatter(goff, [spv * _RB + ul], o_)
                        plsc.addupdate_scatter(cnts, [o_], ones_i,
                                               mask=ul < valid)
                        return c2
                    lax.fori_loop(0, _RB // 16, u_step, 0)

                def gather_start(sp):
                    so = pl.multiple_of(sp * _RB, 8)
                    pltpu.async_copy(ft_hbm.at[gid.at[pl.ds(so, _RB)]],
                                     rows.at[sp], gsem)

                def gather_wait(sp):
                    so = pl.multiple_of(sp * _RB, 8)
                    pltpu.make_async_copy(ft_hbm.at[gid.at[pl.ds(so, _RB)]],
                                          rows.at[sp], gsem).wait()

                def do_sub(s, carry2):
                    sp = s & 1
                    k0 = s * _RB
                    valid = jnp.minimum(cnt - k0, _RB)
                    gather_wait(sp)

                    @pl.when(s + 1 < nsb)
                    def _():
                        unpack(s + 1, 1 - sp)
                        gather_start(1 - sp)

                    spv = lax.broadcast(sp, (16,))

                    def apply_pt(j, c3):
                        jv = lax.broadcast(j, (16,))
                        base = plsc.load_gather(goff, [spv * _RB + jv]) * 128
                        for k in range(8):
                            kl = k * 16 + lanes
                            cur = plsc.load_gather(tbl, [base + kl])
                            val = plsc.load_gather(rows, [spv, jv, kl])
                            plsc.store_scatter(tbl, [base + kl],
                                               jnp.maximum(cur, val))
                        return c3
                    lax.fori_loop(0, valid, apply_pt, 0)
                    return carry2

                nsb = lax.shift_right_logical(cnt + _RB - 1, 7)

                with jax.named_scope("apply"):
                    @pl.when(nsb > 0)
                    def _():
                        unpack(0, 0)
                        gather_start(0)

                    lax.fori_loop(0, nsb, do_sub, 0)
                return carry

            lax.fori_loop(0, _NCH, do_chunk, 0)

            def escan(e, ce):
                c16 = plsc.load_gather(cnts, [e * 16 + lanes])
                m = c16 == 0
                ranks = plsc.cumsum(jnp.where(m, 1, 0))
                plsc.store_scatter(selp, [ce + ranks - 1], e * 16 + lanes,
                                   mask=m)
                return ce + jnp.max(ranks)
            cnt_e = lax.fori_loop(0, _RV // 16, escan, 0)

            def ezero(j, cz):
                ov = plsc.load_gather(selp, [lax.broadcast(j, (16,))]) * 128
                for k in range(8):
                    plsc.store_scatter(tbl, [ov + k * 16 + lanes], zeros_f)
                return cz
            lax.fori_loop(0, cnt_e, ezero, 0)

            def cclip(e, cc):
                c16 = plsc.load_gather(cnts, [e * 16 + lanes])
                plsc.store_scatter(cnts, [e * 16 + lanes],
                                   jnp.maximum(c16, 1))
                return cc
            lax.fori_loop(0, _RV // 16, cclip, 0)

            pltpu.sync_copy(cnts, cnt_hbm.at[pl.ds(b * NUM_VOXELS + lo, _RV)])
            pltpu.sync_copy(
                tbl, vf_hbm.at[pl.ds((b * NUM_VOXELS + lo) * _D, _RV * _D)])

        def task_body(q, cq):
            r = w + _NW * lax.shift_right_logical(q, 2)
            b = q & 3
            process_task(q, r, b, (q + 1) & 3, q == 2 * _B - 1)
            return cq
        lax.fori_loop(0, 2 * _B, task_body, 0)

    return scat


_scatter_kernel = _make_scatter_kernel()


def kernel(features, xyz_coords_for_voxelization):
    B, D, N = features.shape
    idx = _idx_kernel(xyz_coords_for_voxelization.reshape(-1))
    ft = _transpose_features(features).reshape(B * N, D)
    vf_vm, counts = _scatter_kernel(ft, idx)
    vf = jnp.swapaxes(vf_vm.reshape(B, NUM_VOXELS, D), 1, 2)
    return (vf.reshape(B, D, GRID, GRID, GRID),
            idx.reshape(B, N),
            counts.reshape(B, 1, NUM_VOXELS))
